# Initial kernel scaffold; baseline (speedup 1.0000x reference)
import jax
import jax.numpy as jnp
from jax.experimental import pallas as pl


def kernel(x, embed_table, pos_emb, W, b):
    raise NotImplementedError("write your pallas kernel here")


# speedup vs baseline: 1.2749x; 1.2749x over previous
import functools

import jax
import jax.numpy as jnp
from jax import lax
from jax.experimental import pallas as pl
from jax.experimental.pallas import tpu as pltpu
from jax.experimental.pallas import tpu_sc as plsc

SEQ = 200
EMB = 16
BATCH = 4096
FLAT = SEQ * EMB
NROWS = BATCH * SEQ

NUM_CORES = 2
NUM_SUBCORES = 16
NW = NUM_CORES * NUM_SUBCORES

CHUNK = 128
K = 8
G = NROWS // (NW * K * CHUNK)
NCHUNKS = NROWS // CHUNK
CH_PER_W = NCHUNKS // NW


def _sc_gather_body(table_hbm, idx_hbm, out_hbm, idx_v, rows_v, sem):
    wid = lax.axis_index("s") * NUM_CORES + lax.axis_index("c")
    chunk_base = wid * CH_PER_W

    def body(g, carry):
        chunk0 = chunk_base + g * K
        pltpu.sync_copy(idx_hbm.at[pl.ds(chunk0, K)], idx_v)
        handles = []
        for j in range(K):
            handles.append(
                pltpu.async_copy(table_hbm.at[idx_v.at[j]], rows_v.at[j], sem))
        for h in handles:
            h.wait()
        pltpu.sync_copy(rows_v, out_hbm.at[pl.ds(chunk0, K)])
        return carry

    lax.fori_loop(0, G, body, 0)


_sc_gather = pl.kernel(
    _sc_gather_body,
    out_type=jax.ShapeDtypeStruct((NCHUNKS, CHUNK, EMB), jnp.float32),
    mesh=plsc.VectorSubcoreMesh(
        core_axis_name="c", subcore_axis_name="s",
        num_cores=NUM_CORES, num_subcores=NUM_SUBCORES),
    scratch_types=[
        pltpu.VMEM((K, CHUNK), jnp.int32),
        pltpu.VMEM((K, CHUNK, EMB), jnp.float32),
        pltpu.SemaphoreType.DMA,
    ],
    compiler_params=pltpu.CompilerParams(use_tc_tiling_on_sc=False),
)


def _tc_matmul_body(emb_ref, pos_ref, w_ref, b_ref, out_ref):
    acc = jnp.dot(emb_ref[...] + pos_ref[...], w_ref[...],
                  preferred_element_type=jnp.float32)
    out_ref[...] = acc + b_ref[...]


BM = 512


def _tc_matmul(flat, pos_flat, W, b2):
    grid = (BATCH // BM,)
    return pl.pallas_call(
        _tc_matmul_body,
        out_shape=jax.ShapeDtypeStruct((BATCH, 16), jnp.float32),
        grid=grid,
        in_specs=[
            pl.BlockSpec((BM, FLAT), lambda i: (i, 0)),
            pl.BlockSpec((1, FLAT), lambda i: (0, 0)),
            pl.BlockSpec((FLAT, 16), lambda i: (0, 0)),
            pl.BlockSpec((1, 16), lambda i: (0, 0)),
        ],
        out_specs=pl.BlockSpec((BM, 16), lambda i: (i, 0)),
    )(flat, pos_flat, W, b2)


def kernel(x, embed_table, pos_emb, W, b):
    xi = jnp.asarray(x, dtype=jnp.int32)
    idx2 = xi.reshape(NCHUNKS, CHUNK)
    gathered = _sc_gather(embed_table, idx2)
    flat = gathered.reshape(BATCH, FLAT)
    out = _tc_matmul(flat, pos_emb.reshape(1, FLAT), W, b.reshape(1, 16))
    return out


# speedup vs baseline: 1.7572x; 1.3783x over previous
import functools

import jax
import jax.numpy as jnp
from jax import lax
from jax.experimental import pallas as pl
from jax.experimental.pallas import tpu as pltpu
from jax.experimental.pallas import tpu_sc as plsc

SEQ = 200
EMB = 16
BATCH = 4096
FLAT = SEQ * EMB
NROWS = BATCH * SEQ
VOCAB = 1000000
NG = SEQ // 8
QB = BATCH // 16

BK = 8192
BLKS = 123
VPAD = BLKS * BK

NUM_CORES = 2
NUM_SUBCORES = 16
NW = NUM_CORES * NUM_SUBCORES

CHUNK = 128
K = 8
G = NROWS // (NW * K * CHUNK)
NCHUNKS = NROWS // CHUNK
CH_PER_W = NCHUNKS // NW


def _repack_body(tt_ref, out_ref):
    x = tt_ref[...]
    eye = (jax.lax.broadcasted_iota(jnp.int32, (EMB, EMB), 0)
           == jax.lax.broadcasted_iota(jnp.int32, (EMB, EMB), 1)
           ).astype(jnp.float32)
    t = jax.lax.dot_general(x, eye, (((0,), (0,)), ((), ())),
                            preferred_element_type=jnp.float32)
    for k in range(8):
        out_ref[:, 16 * k:16 * (k + 1)] = t[(BK // 8) * k:(BK // 8) * (k + 1), :]


def _repack(table_t):
    return pl.pallas_call(
        _repack_body,
        out_shape=jax.ShapeDtypeStruct((VPAD // 8, 128), jnp.float32),
        grid=(BLKS,),
        in_specs=[pl.BlockSpec((EMB, BK), lambda j: (0, j))],
        out_specs=pl.BlockSpec((BK // 8, 128), lambda j: (j, 0)),
    )(table_t)


def _sc_gather_body(table_hbm, idx_hbm, out_hbm, idx_v, rows_v, sem):
    wid = lax.axis_index("s") * NUM_CORES + lax.axis_index("c")
    chunk_base = wid * CH_PER_W

    def body(g, carry):
        chunk0 = chunk_base + g * K
        c = chunk0 // QB
        q0 = chunk0 % QB
        pltpu.sync_copy(idx_hbm.at[pl.ds(chunk0, K)], idx_v)
        handles = []
        for j in range(K):
            handles.append(
                pltpu.async_copy(table_hbm.at[idx_v.at[j]], rows_v.at[j], sem))
        for h in handles:
            h.wait()
        pltpu.sync_copy(rows_v, out_hbm.at[c, pl.ds(q0, K)])
        return carry

    lax.fori_loop(0, G, body, 0)


_sc_gather = pl.kernel(
    _sc_gather_body,
    out_type=jax.ShapeDtypeStruct((NG, QB, CHUNK, EMB), jnp.float32),
    mesh=plsc.VectorSubcoreMesh(
        core_axis_name="c", subcore_axis_name="s",
        num_cores=NUM_CORES, num_subcores=NUM_SUBCORES),
    scratch_types=[
        pltpu.VMEM((K, CHUNK), jnp.int32),
        pltpu.VMEM((K, CHUNK, EMB), jnp.float32),
        pltpu.SemaphoreType.DMA,
    ],
    compiler_params=pltpu.CompilerParams(use_tc_tiling_on_sc=False),
)


def _tc_matmul_body(g_ref, pos_ref, w_ref, b_ref, out_ref):
    accp = jnp.dot(pos_ref[...], w_ref[...],
                   preferred_element_type=jnp.float32)
    acc = jnp.zeros_like(out_ref)
    for c in range(NG):
        acc = acc + jnp.dot(g_ref[c], w_ref[128 * c:128 * (c + 1), :],
                            preferred_element_type=jnp.float32)
    out_ref[...] = acc + accp + b_ref[...]


BM = 512


def _tc_matmul(g3, pos_flat, W, b2):
    grid = (BATCH // BM,)
    return pl.pallas_call(
        _tc_matmul_body,
        out_shape=jax.ShapeDtypeStruct((BATCH, 16), jnp.float32),
        grid=grid,
        in_specs=[
            pl.BlockSpec((NG, BM, 128), lambda i: (0, i, 0)),
            pl.BlockSpec((1, FLAT), lambda i: (0, 0)),
            pl.BlockSpec((FLAT, 16), lambda i: (0, 0)),
            pl.BlockSpec((1, 16), lambda i: (0, 0)),
        ],
        out_specs=pl.BlockSpec((BM, 16), lambda i: (i, 0)),
    )(g3, pos_flat, W, b2)


def kernel(x, embed_table, pos_emb, W, b):
    xi = jnp.asarray(x, dtype=jnp.int32)
    xs = (xi & ~jnp.int32(BK - 1)) | ((xi & (BK // 8 - 1)) << 3) | \
         ((xi >> 10) & 7)
    idx2 = xs.reshape(BATCH, NG, 8).transpose(1, 0, 2).reshape(NCHUNKS, CHUNK)
    packed = _repack(embed_table.T)
    table_lin = packed.reshape(VPAD, EMB)
    gathered = _sc_gather(table_lin, idx2)
    g3 = gathered.reshape(NG, BATCH, 128)
    out = _tc_matmul(g3, pos_emb.reshape(1, FLAT), W, b.reshape(1, 16))
    return out


# speedup vs baseline: 2.7926x; 1.5893x over previous
import functools

import jax
import jax.numpy as jnp
from jax import lax
from jax.experimental import pallas as pl
from jax.experimental.pallas import tpu as pltpu
from jax.experimental.pallas import tpu_sc as plsc

SEQ = 200
EMB = 16
BATCH = 4096
FLAT = SEQ * EMB
NROWS = BATCH * SEQ
VOCAB = 1000000
NG = SEQ // 8
QB = BATCH // 16

BK = 8192
BLKS = 123
VPAD = BLKS * BK

NUM_CORES = 2
NUM_SUBCORES = 16
NW = NUM_CORES * NUM_SUBCORES

CHUNK = 128
K = 8
G = NROWS // (NW * K * CHUNK)
NCHUNKS = NROWS // CHUNK
CH_PER_W = NCHUNKS // NW


def _repack_body(tt_ref, out_ref):
    x = tt_ref[...]
    y = jnp.concatenate(
        [x[:, (BK // 8) * k:(BK // 8) * (k + 1)] for k in range(8)], axis=0)
    out_ref[...] = jnp.swapaxes(y, 0, 1)


def _repack(table_t):
    return pl.pallas_call(
        _repack_body,
        out_shape=jax.ShapeDtypeStruct((VPAD // 8, 128), jnp.float32),
        grid=(BLKS,),
        in_specs=[pl.BlockSpec((EMB, BK), lambda j: (0, j))],
        out_specs=pl.BlockSpec((BK // 8, 128), lambda j: (j, 0)),
    )(table_t)


def _sc_gather_body(table_hbm, idx_hbm, out_hbm, idx_v, rows_v, sem):
    wid = lax.axis_index("s") * NUM_CORES + lax.axis_index("c")
    chunk_base = wid * CH_PER_W

    def body(g, carry):
        chunk0 = chunk_base + g * K
        c = chunk0 // QB
        q0 = chunk0 % QB
        pltpu.sync_copy(idx_hbm.at[pl.ds(chunk0, K)], idx_v)
        handles = []
        for j in range(K):
            handles.append(
                pltpu.async_copy(table_hbm.at[idx_v.at[j]], rows_v.at[j], sem))
        for h in handles:
            h.wait()
        pltpu.sync_copy(rows_v, out_hbm.at[c, pl.ds(q0, K)])
        return carry

    lax.fori_loop(0, G, body, 0)


_sc_gather = pl.kernel(
    _sc_gather_body,
    out_type=jax.ShapeDtypeStruct((NG, QB, CHUNK, EMB), jnp.float32),
    mesh=plsc.VectorSubcoreMesh(
        core_axis_name="c", subcore_axis_name="s",
        num_cores=NUM_CORES, num_subcores=NUM_SUBCORES),
    scratch_types=[
        pltpu.VMEM((K, CHUNK), jnp.int32),
        pltpu.VMEM((K, CHUNK, EMB), jnp.float32),
        pltpu.SemaphoreType.DMA,
    ],
    compiler_params=pltpu.CompilerParams(use_tc_tiling_on_sc=False),
)


def _tc_matmul_body(g_ref, pos_ref, w_ref, b_ref, out_ref):
    accp = jnp.dot(pos_ref[...], w_ref[...],
                   preferred_element_type=jnp.float32)
    acc = jnp.zeros_like(out_ref)
    for c in range(NG):
        acc = acc + jnp.dot(g_ref[c], w_ref[128 * c:128 * (c + 1), :],
                            preferred_element_type=jnp.float32)
    out_ref[...] = acc + accp + b_ref[...]


BM = 512


def _tc_matmul(g3, pos_flat, W, b2):
    grid = (BATCH // BM,)
    return pl.pallas_call(
        _tc_matmul_body,
        out_shape=jax.ShapeDtypeStruct((BATCH, 16), jnp.float32),
        grid=grid,
        in_specs=[
            pl.BlockSpec((NG, BM, 128), lambda i: (0, i, 0)),
            pl.BlockSpec((1, FLAT), lambda i: (0, 0)),
            pl.BlockSpec((FLAT, 16), lambda i: (0, 0)),
            pl.BlockSpec((1, 16), lambda i: (0, 0)),
        ],
        out_specs=pl.BlockSpec((BM, 16), lambda i: (i, 0)),
    )(g3, pos_flat, W, b2)


def kernel(x, embed_table, pos_emb, W, b):
    xi = jnp.asarray(x, dtype=jnp.int32)
    xs = (xi & ~jnp.int32(BK - 1)) | ((xi & (BK // 8 - 1)) << 3) | \
         ((xi >> 10) & 7)
    idx2 = xs.reshape(BATCH, NG, 8).transpose(1, 0, 2).reshape(NCHUNKS, CHUNK)
    packed = _repack(embed_table.T)
    table_lin = packed.reshape(VPAD, EMB)
    gathered = _sc_gather(table_lin, idx2)
    g3 = gathered.reshape(NG, BATCH, 128)
    out = _tc_matmul(g3, pos_emb.reshape(1, FLAT), W, b.reshape(1, 16))
    return out


# speedup vs baseline: 3.1042x; 1.1116x over previous
import functools

import jax
import jax.numpy as jnp
from jax import lax
from jax.experimental import pallas as pl
from jax.experimental.pallas import tpu as pltpu
from jax.experimental.pallas import tpu_sc as plsc

SEQ = 200
EMB = 16
BATCH = 4096
FLAT = SEQ * EMB
NROWS = BATCH * SEQ
VOCAB = 1000000
NG = SEQ // 8
QB = BATCH // 16

BK = 16384
BLKS = 62
RSH = (BK // 8).bit_length() - 1
VPAD = BLKS * BK

NUM_CORES = 2
NUM_SUBCORES = 16
NW = NUM_CORES * NUM_SUBCORES

CHUNK = 128
K = 8
G = NROWS // (NW * K * CHUNK)
NCHUNKS = NROWS // CHUNK
CH_PER_W = NCHUNKS // NW


def _repack_body(tt_ref, out_ref):
    x = tt_ref[...]
    y = jnp.concatenate(
        [x[:, (BK // 8) * k:(BK // 8) * (k + 1)] for k in range(8)], axis=0)
    out_ref[...] = jnp.swapaxes(y, 0, 1)


def _repack(table_t):
    return pl.pallas_call(
        _repack_body,
        out_shape=jax.ShapeDtypeStruct((VPAD // 8, 128), jnp.float32),
        grid=(BLKS,),
        in_specs=[pl.BlockSpec((EMB, BK), lambda j: (0, j))],
        out_specs=pl.BlockSpec((BK // 8, 128), lambda j: (j, 0)),
    )(table_t)


def _sc_gather_body(table_hbm, idx_hbm, out_hbm, idx_v, rows_v, sem):
    wid = lax.axis_index("s") * NUM_CORES + lax.axis_index("c")
    chunk_base = wid * CH_PER_W

    def body(g, carry):
        chunk0 = chunk_base + g * K
        c = chunk0 // QB
        q0 = chunk0 % QB
        pltpu.sync_copy(idx_hbm.at[pl.ds(chunk0, K)], idx_v)
        handles = []
        for j in range(K):
            handles.append(
                pltpu.async_copy(table_hbm.at[idx_v.at[j]], rows_v.at[j], sem))
        for h in handles:
            h.wait()
        pltpu.sync_copy(rows_v, out_hbm.at[c, pl.ds(q0, K)])
        return carry

    lax.fori_loop(0, G, body, 0)


_sc_gather = pl.kernel(
    _sc_gather_body,
    out_type=jax.ShapeDtypeStruct((NG, QB, CHUNK, EMB), jnp.float32),
    mesh=plsc.VectorSubcoreMesh(
        core_axis_name="c", subcore_axis_name="s",
        num_cores=NUM_CORES, num_subcores=NUM_SUBCORES),
    scratch_types=[
        pltpu.VMEM((K, CHUNK), jnp.int32),
        pltpu.VMEM((K, CHUNK, EMB), jnp.float32),
        pltpu.SemaphoreType.DMA,
    ],
    compiler_params=pltpu.CompilerParams(use_tc_tiling_on_sc=False),
)


def _tc_matmul_body(g_ref, pos_ref, w_ref, b_ref, out_ref):
    accp = jnp.dot(pos_ref[...], w_ref[...],
                   preferred_element_type=jnp.float32)
    acc = jnp.zeros_like(out_ref)
    for c in range(NG):
        acc = acc + jnp.dot(g_ref[c], w_ref[128 * c:128 * (c + 1), :],
                            preferred_element_type=jnp.float32)
    out_ref[...] = acc + accp + b_ref[...]


BM = 512


def _tc_matmul(g3, pos_flat, W, b2):
    grid = (BATCH // BM,)
    return pl.pallas_call(
        _tc_matmul_body,
        out_shape=jax.ShapeDtypeStruct((BATCH, 16), jnp.float32),
        grid=grid,
        in_specs=[
            pl.BlockSpec((NG, BM, 128), lambda i: (0, i, 0)),
            pl.BlockSpec((1, FLAT), lambda i: (0, 0)),
            pl.BlockSpec((FLAT, 16), lambda i: (0, 0)),
            pl.BlockSpec((1, 16), lambda i: (0, 0)),
        ],
        out_specs=pl.BlockSpec((BM, 16), lambda i: (i, 0)),
    )(g3, pos_flat, W, b2)


def kernel(x, embed_table, pos_emb, W, b):
    xi = jnp.asarray(x, dtype=jnp.int32)
    xs = (xi & ~jnp.int32(BK - 1)) | ((xi & (BK // 8 - 1)) << 3) | \
         ((xi >> RSH) & 7)
    idx2 = xs.reshape(BATCH, NG, 8).transpose(1, 0, 2).reshape(NCHUNKS, CHUNK)
    packed = _repack(embed_table.T)
    table_lin = packed.reshape(VPAD, EMB)
    gathered = _sc_gather(table_lin, idx2)
    g3 = gathered.reshape(NG, BATCH, 128)
    out = _tc_matmul(g3, pos_emb.reshape(1, FLAT), W, b.reshape(1, 16))
    return out


# speedup vs baseline: 4.0059x; 1.2905x over previous
import functools

import jax
import jax.numpy as jnp
from jax import lax
from jax.experimental import pallas as pl
from jax.experimental.pallas import tpu as pltpu
from jax.experimental.pallas import tpu_sc as plsc

SEQ = 200
EMB = 16
BATCH = 4096
FLAT = SEQ * EMB
NROWS = BATCH * SEQ
VOCAB = 1000000
NG = SEQ // 8
QB = BATCH // 16

BK = 16384
BLKS = 62
RSH = (BK // 8).bit_length() - 1
VPAD = BLKS * BK

NUM_CORES = 2
NUM_SUBCORES = 16
NW = NUM_CORES * NUM_SUBCORES

CHUNK = 128
K = 8
G = NROWS // (NW * K * CHUNK)
NCHUNKS = NROWS // CHUNK
CH_PER_W = NCHUNKS // NW


def _repack_body(tt_ref, out_ref):
    x = tt_ref[...]
    y = jnp.concatenate(
        [x[:, (BK // 8) * k:(BK // 8) * (k + 1)] for k in range(8)], axis=0)
    out_ref[...] = jnp.swapaxes(y, 0, 1)


def _repack(table_t):
    return pl.pallas_call(
        _repack_body,
        out_shape=jax.ShapeDtypeStruct((VPAD // 8, 128), jnp.float32),
        grid=(BLKS,),
        in_specs=[pl.BlockSpec((EMB, BK), lambda j: (0, j))],
        out_specs=pl.BlockSpec((BK // 8, 128), lambda j: (j, 0)),
    )(table_t)


def _sc_gather_body(table_hbm, x_hbm, out_hbm, xloc, idx_v, rows_v, sem):
    wid = lax.axis_index("s") * NUM_CORES + lax.axis_index("c")
    q0 = wid * K
    pltpu.sync_copy(x_hbm.at[pl.ds(wid * 128, 128)], xloc)
    lane = lax.iota(jnp.int32, 16)
    rowpat = lane >> 3
    colpat = lane & 7

    def body(c, carry):
        for j in range(K):
            for v in range(8):
                rvec = (16 * j + 2 * v) + rowpat
                cvec = 8 * c + colpat
                vals = plsc.load_gather(xloc, [rvec, cvec])
                sw = ((vals & ~jnp.int32(BK - 1))
                      | ((vals & (BK // 8 - 1)) << 3)
                      | ((vals >> RSH) & 7))
                idx_v[j, pl.ds(16 * v, 16)] = sw
        handles = []
        for j in range(K):
            handles.append(
                pltpu.async_copy(table_hbm.at[idx_v.at[j]], rows_v.at[j], sem))
        for h in handles:
            h.wait()
        pltpu.sync_copy(rows_v, out_hbm.at[c, pl.ds(q0, K)])
        return carry

    lax.fori_loop(0, NG, body, 0)


_sc_gather = pl.kernel(
    _sc_gather_body,
    out_type=jax.ShapeDtypeStruct((NG, QB, CHUNK, EMB), jnp.float32),
    mesh=plsc.VectorSubcoreMesh(
        core_axis_name="c", subcore_axis_name="s",
        num_cores=NUM_CORES, num_subcores=NUM_SUBCORES),
    scratch_types=[
        pltpu.VMEM((128, SEQ), jnp.int32),
        pltpu.VMEM((K, CHUNK), jnp.int32),
        pltpu.VMEM((K, CHUNK, EMB), jnp.float32),
        pltpu.SemaphoreType.DMA,
    ],
    compiler_params=pltpu.CompilerParams(use_tc_tiling_on_sc=False,
                                         needs_layout_passes=False),
)


def _tc_matmul_body(g_ref, pos_ref, w_ref, b_ref, out_ref):
    accp = jnp.dot(pos_ref[...], w_ref[...],
                   preferred_element_type=jnp.float32)
    acc = jnp.zeros_like(out_ref)
    for c in range(NG):
        acc = acc + jnp.dot(g_ref[c], w_ref[128 * c:128 * (c + 1), :],
                            preferred_element_type=jnp.float32)
    out_ref[...] = acc + accp + b_ref[...]


BM = 512


def _tc_matmul(g3, pos_flat, W, b2):
    grid = (BATCH // BM,)
    return pl.pallas_call(
        _tc_matmul_body,
        out_shape=jax.ShapeDtypeStruct((BATCH, 16), jnp.float32),
        grid=grid,
        in_specs=[
            pl.BlockSpec((NG, BM, 128), lambda i: (0, i, 0)),
            pl.BlockSpec((1, FLAT), lambda i: (0, 0)),
            pl.BlockSpec((FLAT, 16), lambda i: (0, 0)),
            pl.BlockSpec((1, 16), lambda i: (0, 0)),
        ],
        out_specs=pl.BlockSpec((BM, 16), lambda i: (i, 0)),
    )(g3, pos_flat, W, b2)


def kernel(x, embed_table, pos_emb, W, b):
    xi = jnp.asarray(x, dtype=jnp.int32)
    packed = _repack(embed_table.T)
    table_lin = packed.reshape(VPAD, EMB)
    gathered = _sc_gather(table_lin, xi)
    g3 = gathered.reshape(NG, BATCH, 128)
    out = _tc_matmul(g3, pos_emb.reshape(1, FLAT), W, b.reshape(1, 16))
    return out


# speedup vs baseline: 4.4598x; 1.1133x over previous
import functools

import jax
import jax.numpy as jnp
from jax import lax
from jax.experimental import pallas as pl
from jax.experimental.pallas import tpu as pltpu
from jax.experimental.pallas import tpu_sc as plsc

SEQ = 200
EMB = 16
BATCH = 4096
FLAT = SEQ * EMB
NROWS = BATCH * SEQ
VOCAB = 1000000
NG = SEQ // 8
QB = BATCH // 16

BK = 32768
BLKS = 31
RSH = (BK // 8).bit_length() - 1
VPAD = BLKS * BK

NUM_CORES = 2
NUM_SUBCORES = 16
NW = NUM_CORES * NUM_SUBCORES

CHUNK = 128
K = 8
G = NROWS // (NW * K * CHUNK)
NCHUNKS = NROWS // CHUNK
CH_PER_W = NCHUNKS // NW


def _repack_body(tt_ref, out_ref):
    x = tt_ref[...]
    y = jnp.concatenate(
        [x[:, (BK // 8) * k:(BK // 8) * (k + 1)] for k in range(8)], axis=0)
    out_ref[...] = jnp.swapaxes(y, 0, 1)


def _repack(table_t):
    return pl.pallas_call(
        _repack_body,
        out_shape=jax.ShapeDtypeStruct((VPAD // 8, 128), jnp.float32),
        grid=(BLKS,),
        in_specs=[pl.BlockSpec((EMB, BK), lambda j: (0, j))],
        out_specs=pl.BlockSpec((BK // 8, 128), lambda j: (j, 0)),
    )(table_t)


def _sc_gather_body(table_hbm, x_hbm, out_hbm, xloc, idx_v, rows_v, sem):
    wid = lax.axis_index("s") * NUM_CORES + lax.axis_index("c")
    q0 = wid * K
    pltpu.sync_copy(x_hbm.at[pl.ds(wid * 128, 128)], xloc)
    lane = lax.iota(jnp.int32, 16)
    rowpat = lane >> 3
    colpat = lane & 7

    def body(c, carry):
        for j in range(K):
            for v in range(8):
                rvec = (16 * j + 2 * v) + rowpat
                cvec = 8 * c + colpat
                vals = plsc.load_gather(xloc, [rvec, cvec])
                sw = ((vals & ~jnp.int32(BK - 1))
                      | ((vals & (BK // 8 - 1)) << 3)
                      | ((vals >> RSH) & 7))
                idx_v[j, pl.ds(16 * v, 16)] = sw
        handles = []
        for j in range(K):
            handles.append(
                pltpu.async_copy(table_hbm.at[idx_v.at[j]], rows_v.at[j], sem))
        for h in handles:
            h.wait()
        pltpu.sync_copy(rows_v, out_hbm.at[c, pl.ds(q0, K)])
        return carry

    lax.fori_loop(0, NG, body, 0)


_sc_gather = pl.kernel(
    _sc_gather_body,
    out_type=jax.ShapeDtypeStruct((NG, QB, CHUNK, EMB), jnp.float32),
    mesh=plsc.VectorSubcoreMesh(
        core_axis_name="c", subcore_axis_name="s",
        num_cores=NUM_CORES, num_subcores=NUM_SUBCORES),
    scratch_types=[
        pltpu.VMEM((128, SEQ), jnp.int32),
        pltpu.VMEM((K, CHUNK), jnp.int32),
        pltpu.VMEM((K, CHUNK, EMB), jnp.float32),
        pltpu.SemaphoreType.DMA,
    ],
    compiler_params=pltpu.CompilerParams(use_tc_tiling_on_sc=False,
                                         needs_layout_passes=False),
)


def _tc_matmul_body(g_ref, pos_ref, w_ref, b_ref, out_ref):
    accp = jnp.dot(pos_ref[...], w_ref[...],
                   preferred_element_type=jnp.float32)
    acc = jnp.zeros_like(out_ref)
    for c in range(NG):
        acc = acc + jnp.dot(g_ref[c], w_ref[128 * c:128 * (c + 1), :],
                            preferred_element_type=jnp.float32)
    out_ref[...] = acc + accp + b_ref[...]


BM = 512


def _tc_matmul(g3, pos_flat, W, b2):
    grid = (BATCH // BM,)
    return pl.pallas_call(
        _tc_matmul_body,
        out_shape=jax.ShapeDtypeStruct((BATCH, 16), jnp.float32),
        grid=grid,
        in_specs=[
            pl.BlockSpec((NG, BM, 128), lambda i: (0, i, 0)),
            pl.BlockSpec((1, FLAT), lambda i: (0, 0)),
            pl.BlockSpec((FLAT, 16), lambda i: (0, 0)),
            pl.BlockSpec((1, 16), lambda i: (0, 0)),
        ],
        out_specs=pl.BlockSpec((BM, 16), lambda i: (i, 0)),
    )(g3, pos_flat, W, b2)


def kernel(x, embed_table, pos_emb, W, b):
    xi = jnp.asarray(x, dtype=jnp.int32)
    packed = _repack(embed_table.T)
    table_lin = packed.reshape(VPAD, EMB)
    gathered = _sc_gather(table_lin, xi)
    g3 = gathered.reshape(NG, BATCH, 128)
    out = _tc_matmul(g3, pos_emb.reshape(1, FLAT), W, b.reshape(1, 16))
    return out


# speedup vs baseline: 5.1566x; 1.1563x over previous
import functools

import jax
import jax.numpy as jnp
from jax import lax
from jax.experimental import pallas as pl
from jax.experimental.pallas import tpu as pltpu
from jax.experimental.pallas import tpu_sc as plsc

SEQ = 200
EMB = 16
BATCH = 4096
FLAT = SEQ * EMB
NROWS = BATCH * SEQ
VOCAB = 1000000
NG = SEQ // 8
QB = BATCH // 16

BK = 32768
BLKS = 31
RSH = (BK // 8).bit_length() - 1
VPAD = BLKS * BK

NUM_CORES = 2
NUM_SUBCORES = 16
NW = NUM_CORES * NUM_SUBCORES

CHUNK = 128
K = 8
G = NROWS // (NW * K * CHUNK)
NCHUNKS = NROWS // CHUNK
CH_PER_W = NCHUNKS // NW


def _repack_body(tt_ref, out_ref):
    x = tt_ref[...]
    y = jnp.concatenate(
        [x[:, (BK // 8) * k:(BK // 8) * (k + 1)] for k in range(8)], axis=0)
    out_ref[...] = jnp.swapaxes(y, 0, 1)


def _repack(table_t):
    return pl.pallas_call(
        _repack_body,
        out_shape=jax.ShapeDtypeStruct((VPAD // 8, 128), jnp.float32),
        grid=(BLKS,),
        in_specs=[pl.BlockSpec((EMB, BK), lambda j: (0, j))],
        out_specs=pl.BlockSpec((BK // 8, 128), lambda j: (j, 0)),
    )(table_t)


def _sc_idx_body(x_hbm, out_hbm, xloc, idx_v):
    wid = lax.axis_index("s") * NUM_CORES + lax.axis_index("c")
    pltpu.sync_copy(x_hbm.at[pl.ds(wid * 128, 128)], xloc)
    lane = lax.iota(jnp.int32, 16)
    rowpat = lane >> 3
    colpat = lane & 7

    def body(c, carry):
        for j in range(K):
            for v in range(8):
                rvec = (16 * j + 2 * v) + rowpat
                cvec = 8 * c + colpat
                vals = plsc.load_gather(xloc, [rvec, cvec])
                sw = ((vals & ~jnp.int32(BK - 1))
                      | ((vals & (BK // 8 - 1)) << 3)
                      | ((vals >> RSH) & 7))
                idx_v[c, j, pl.ds(16 * v, 16)] = sw
        return carry

    lax.fori_loop(0, NG, body, 0)
    pltpu.sync_copy(idx_v, out_hbm.at[wid])


_sc_idx = pl.kernel(
    _sc_idx_body,
    out_type=jax.ShapeDtypeStruct((NW, NG, K, CHUNK), jnp.int32),
    mesh=plsc.VectorSubcoreMesh(
        core_axis_name="c", subcore_axis_name="s",
        num_cores=NUM_CORES, num_subcores=NUM_SUBCORES),
    scratch_types=[
        pltpu.VMEM((128, SEQ), jnp.int32),
        pltpu.VMEM((NG, K, CHUNK), jnp.int32),
    ],
    compiler_params=pltpu.CompilerParams(use_tc_tiling_on_sc=False,
                                         needs_layout_passes=False),
)


def _sc_gather_body(table_hbm, idx_hbm, out_hbm, idx_v, rows_v,
                    gsem0, gsem1, wsem0, wsem1):
    wid = lax.axis_index("s") * NUM_CORES + lax.axis_index("c")
    q0 = wid * K
    gsem = (gsem0, gsem1)
    wsem = (wsem0, wsem1)
    pltpu.sync_copy(idx_hbm.at[wid], idx_v)

    def fire(c, h):
        for j in range(K):
            pltpu.async_copy(table_hbm.at[idx_v.at[c, j]], rows_v.at[h, j],
                             gsem[h])

    def wait_gather(h):
        for j in range(K):
            pltpu.make_async_copy(table_hbm.at[idx_v.at[0, j]],
                                  rows_v.at[h, j], gsem[h]).wait()

    def write(c, h):
        pltpu.async_copy(rows_v.at[h], out_hbm.at[c, pl.ds(q0, K)], wsem[h])

    def wait_write(h):
        pltpu.make_async_copy(rows_v.at[h], out_hbm.at[0, pl.ds(q0, K)],
                              wsem[h]).wait()

    fire(0, 0)
    fire(1, 1)

    def body(p, carry):
        for h in range(2):
            cc = 2 * p + h
            wait_gather(h)
            write(cc, h)

            @pl.when(cc <= NG - 3)
            def _():
                wait_write(h)
                fire(cc + 2, h)
        return carry

    lax.fori_loop(0, (NG - 1) // 2, body, 0)
    wait_gather(0)
    write(NG - 1, 0)
    wait_write(0)
    wait_write(1)


_sc_gather = pl.kernel(
    _sc_gather_body,
    out_type=jax.ShapeDtypeStruct((NG, QB, CHUNK, EMB), jnp.float32),
    mesh=plsc.VectorSubcoreMesh(
        core_axis_name="c", subcore_axis_name="s",
        num_cores=NUM_CORES, num_subcores=NUM_SUBCORES),
    scratch_types=[
        pltpu.VMEM((NG, K, CHUNK), jnp.int32),
        pltpu.VMEM((2, K, CHUNK, EMB), jnp.float32),
        pltpu.SemaphoreType.DMA,
        pltpu.SemaphoreType.DMA,
        pltpu.SemaphoreType.DMA,
        pltpu.SemaphoreType.DMA,
    ],
    compiler_params=pltpu.CompilerParams(use_tc_tiling_on_sc=False,
                                         needs_layout_passes=False),
)


def _tc_matmul_body(g_ref, pos_ref, w_ref, b_ref, out_ref):
    accp = jnp.dot(pos_ref[...], w_ref[...],
                   preferred_element_type=jnp.float32)
    acc = jnp.zeros_like(out_ref)
    for c in range(NG):
        acc = acc + jnp.dot(g_ref[c], w_ref[128 * c:128 * (c + 1), :],
                            preferred_element_type=jnp.float32)
    out_ref[...] = acc + accp + b_ref[...]


BM = 512


def _tc_matmul(g3, pos_flat, W, b2):
    grid = (BATCH // BM,)
    return pl.pallas_call(
        _tc_matmul_body,
        out_shape=jax.ShapeDtypeStruct((BATCH, 16), jnp.float32),
        grid=grid,
        in_specs=[
            pl.BlockSpec((NG, BM, 128), lambda i: (0, i, 0)),
            pl.BlockSpec((1, FLAT), lambda i: (0, 0)),
            pl.BlockSpec((FLAT, 16), lambda i: (0, 0)),
            pl.BlockSpec((1, 16), lambda i: (0, 0)),
        ],
        out_specs=pl.BlockSpec((BM, 16), lambda i: (i, 0)),
    )(g3, pos_flat, W, b2)


def kernel(x, embed_table, pos_emb, W, b):
    xi = jnp.asarray(x, dtype=jnp.int32)
    idxall = _sc_idx(xi)
    packed = _repack(embed_table.T)
    table_lin = packed.reshape(VPAD, EMB)
    gathered = _sc_gather(table_lin, idxall)
    g3 = gathered.reshape(NG, BATCH, 128)
    out = _tc_matmul(g3, pos_emb.reshape(1, FLAT), W, b.reshape(1, 16))
    return out


# speedup vs baseline: 5.5155x; 1.0696x over previous
import functools

import jax
import jax.numpy as jnp
from jax import lax
from jax.experimental import pallas as pl
from jax.experimental.pallas import tpu as pltpu
from jax.experimental.pallas import tpu_sc as plsc

SEQ = 200
EMB = 16
BATCH = 4096
FLAT = SEQ * EMB
NROWS = BATCH * SEQ
VOCAB = 1000000
NG = SEQ // 8
QB = BATCH // 16

BK = 32768
BLKS = 31
RSH = (BK // 8).bit_length() - 1
VPAD = BLKS * BK

NUM_CORES = 2
NUM_SUBCORES = 16
NW = NUM_CORES * NUM_SUBCORES

CHUNK = 128
K = 8
G = NROWS // (NW * K * CHUNK)
NCHUNKS = NROWS // CHUNK
CH_PER_W = NCHUNKS // NW


def _repack_body(tt_ref, out_ref):
    x = tt_ref[...]
    y = jnp.concatenate(
        [x[:, (BK // 8) * k:(BK // 8) * (k + 1)] for k in range(8)], axis=0)
    out_ref[...] = jnp.swapaxes(y, 0, 1)


def _repack(table_t):
    return pl.pallas_call(
        _repack_body,
        out_shape=jax.ShapeDtypeStruct((VPAD // 8, 128), jnp.float32),
        grid=(BLKS,),
        in_specs=[pl.BlockSpec((EMB, BK), lambda j: (0, j))],
        out_specs=pl.BlockSpec((BK // 8, 128), lambda j: (j, 0)),
    )(table_t)


def _sc_idx_body(xt_hbm, out_hbm, xloc, idx_v):
    wid = lax.axis_index("s") * NUM_CORES + lax.axis_index("c")
    pltpu.sync_copy(xt_hbm.at[:, pl.ds(wid * 128, 128)], xloc)
    lane = lax.iota(jnp.int32, 16)
    rowpat = lane >> 3
    colpat = lane & 7

    def body(c, carry):
        for j in range(K):
            for v in range(8):
                rvec = 8 * c + colpat
                cvec = (16 * j + 2 * v) + rowpat
                vals = plsc.load_gather(xloc, [rvec, cvec])
                sw = ((vals & ~jnp.int32(BK - 1))
                      | ((vals & (BK // 8 - 1)) << 3)
                      | ((vals >> RSH) & 7))
                idx_v[c, j, pl.ds(16 * v, 16)] = sw
        return carry

    lax.fori_loop(0, NG, body, 0)
    pltpu.sync_copy(idx_v, out_hbm.at[wid])


_sc_idx = pl.kernel(
    _sc_idx_body,
    out_type=jax.ShapeDtypeStruct((NW, NG, K, CHUNK), jnp.int32),
    mesh=plsc.VectorSubcoreMesh(
        core_axis_name="c", subcore_axis_name="s",
        num_cores=NUM_CORES, num_subcores=NUM_SUBCORES),
    scratch_types=[
        pltpu.VMEM((SEQ, 128), jnp.int32),
        pltpu.VMEM((NG, K, CHUNK), jnp.int32),
    ],
    compiler_params=pltpu.CompilerParams(use_tc_tiling_on_sc=False,
                                         needs_layout_passes=False),
)


def _sc_gather_body(table_hbm, idx_hbm, out_hbm, idx_v, rows_v,
                    gsem0, gsem1, wsem0, wsem1):
    wid = lax.axis_index("s") * NUM_CORES + lax.axis_index("c")
    q0 = wid * K
    gsem = (gsem0, gsem1)
    wsem = (wsem0, wsem1)
    pltpu.sync_copy(idx_hbm.at[wid], idx_v)

    def fire(c, h):
        for j in range(K):
            pltpu.async_copy(table_hbm.at[idx_v.at[c, j]], rows_v.at[h, j],
                             gsem[h])

    def wait_gather(h):
        for j in range(K):
            pltpu.make_async_copy(table_hbm.at[idx_v.at[0, j]],
                                  rows_v.at[h, j], gsem[h]).wait()

    def write(c, h):
        pltpu.async_copy(rows_v.at[h], out_hbm.at[c, pl.ds(q0, K)], wsem[h])

    def wait_write(h):
        pltpu.make_async_copy(rows_v.at[h], out_hbm.at[0, pl.ds(q0, K)],
                              wsem[h]).wait()

    fire(0, 0)
    fire(1, 1)

    def body(p, carry):
        for h in range(2):
            cc = 2 * p + h
            wait_gather(h)
            write(cc, h)

            @pl.when(cc <= NG - 3)
            def _():
                wait_write(h)
                fire(cc + 2, h)
        return carry

    lax.fori_loop(0, (NG - 1) // 2, body, 0)
    wait_gather(0)
    write(NG - 1, 0)
    wait_write(0)
    wait_write(1)


_sc_gather = pl.kernel(
    _sc_gather_body,
    out_type=jax.ShapeDtypeStruct((NG, QB, CHUNK, EMB), jnp.float32),
    mesh=plsc.VectorSubcoreMesh(
        core_axis_name="c", subcore_axis_name="s",
        num_cores=NUM_CORES, num_subcores=NUM_SUBCORES),
    scratch_types=[
        pltpu.VMEM((NG, K, CHUNK), jnp.int32),
        pltpu.VMEM((2, K, CHUNK, EMB), jnp.float32),
        pltpu.SemaphoreType.DMA,
        pltpu.SemaphoreType.DMA,
        pltpu.SemaphoreType.DMA,
        pltpu.SemaphoreType.DMA,
    ],
    compiler_params=pltpu.CompilerParams(use_tc_tiling_on_sc=False,
                                         needs_layout_passes=False),
)


def _tc_matmul_body(g_ref, pos_ref, w_ref, b_ref, out_ref):
    accp = jnp.dot(pos_ref[...], w_ref[...],
                   preferred_element_type=jnp.float32)
    acc = jnp.zeros((BM, 16), jnp.float32)
    for c in range(NG):
        acc = acc + jnp.dot(g_ref[c], w_ref[128 * c:128 * (c + 1), :],
                            preferred_element_type=jnp.float32)
    res = acc + accp + b_ref[...]
    out_ref[...] = jnp.swapaxes(res, 0, 1)


BM = 512


def _tc_matmul(g3, pos_flat, W, b2):
    grid = (BATCH // BM,)
    return pl.pallas_call(
        _tc_matmul_body,
        out_shape=jax.ShapeDtypeStruct((16, BATCH), jnp.float32),
        grid=grid,
        in_specs=[
            pl.BlockSpec((NG, BM, 128), lambda i: (0, i, 0)),
            pl.BlockSpec((1, FLAT), lambda i: (0, 0)),
            pl.BlockSpec((FLAT, 16), lambda i: (0, 0)),
            pl.BlockSpec((1, 16), lambda i: (0, 0)),
        ],
        out_specs=pl.BlockSpec((16, BM), lambda i: (0, i)),
    )(g3, pos_flat, W, b2)


def kernel(x, embed_table, pos_emb, W, b):
    xi = jnp.asarray(x, dtype=jnp.int32)
    idxall = _sc_idx(xi.T)
    packed = _repack(embed_table.T)
    table_lin = packed.reshape(VPAD, EMB)
    gathered = _sc_gather(table_lin, idxall)
    g3 = gathered.reshape(NG, BATCH, 128)
    out = _tc_matmul(g3, pos_emb.reshape(1, FLAT), W, b.reshape(1, 16))
    return out.T
